# baseline (speedup 1.0000x reference)
import functools

import jax
import jax.numpy as jnp
from jax import lax
from jax.experimental import pallas as pl
from jax.experimental.pallas import tpu as pltpu
from jax.experimental.pallas import tpu_sc as plsc


def _sc_gather(table2, idx_all, n_rows, NW):
    W = table2.shape[1]
    rpw = n_rows // NW
    CH = 128
    NBUF = 4
    n_grp = rpw // (NBUF * CH)

    mesh = plsc.VectorSubcoreMesh(core_axis_name="c", subcore_axis_name="s")

    @functools.partial(
        pl.kernel,
        mesh=mesh,
        out_type=jax.ShapeDtypeStruct((n_rows, W), jnp.float32),
        scratch_types=[
            pltpu.VMEM((rpw,), jnp.int32),
            pltpu.VMEM((NBUF * CH, W), jnp.float32),
            pltpu.SemaphoreType.DMA,
        ],
    )
    def gather_k(table_hbm, idx_hbm, out_hbm, idx_v, rows_v, sem):
        wid = lax.axis_index("s") * 2 + lax.axis_index("c")
        pltpu.sync_copy(idx_hbm.at[wid], idx_v)
        for g in range(n_grp):
            cps = []
            for j in range(NBUF):
                c = g * NBUF + j
                cps.append(
                    pltpu.async_copy(
                        table_hbm.at[idx_v.at[pl.ds(c * CH, CH)]],
                        rows_v.at[pl.ds(j * CH, CH)],
                        sem,
                    )
                )
            for cp in cps:
                cp.wait()
            for j in range(NBUF):
                c = g * NBUF + j
                pltpu.sync_copy(
                    rows_v.at[pl.ds(j * CH, CH)],
                    out_hbm.at[pl.ds(wid * rpw + c * CH, CH)],
                )

    return gather_k(table2, idx_all)


def _tc_mlp(emb, par, W1, b1, W2, b2, B, L, H, E, S):
    D = W1.shape[0]

    def mlp_k(emb_ref, par_ref, W1_ref, b1_ref, W2_ref, b2_ref, out_ref):
        w1 = W1_ref[...].astype(jnp.bfloat16)
        m = None
        for l in range(L):
            e = emb_ref[l]
            p = par_ref[l]
            es = jnp.where(p > 0.5, e[:, D:], e[:, :D])
            z = jnp.dot(
                es.astype(jnp.bfloat16), w1, preferred_element_type=jnp.float32
            )
            m = z if m is None else jnp.maximum(m, z)
        m = jnp.maximum(m + b1_ref[...], 0.0)
        out_ref[...] = (
            jnp.dot(
                m.astype(jnp.bfloat16),
                W2_ref[...].astype(jnp.bfloat16),
                preferred_element_type=jnp.float32,
            )
            + b2_ref[...]
        )

    return pl.pallas_call(
        mlp_k,
        grid=(B // S,),
        in_specs=[
            pl.BlockSpec((L, S, 2 * D), lambda i: (0, i, 0)),
            pl.BlockSpec((L, S, 1), lambda i: (0, i, 0)),
            pl.BlockSpec((D, H), lambda i: (0, 0)),
            pl.BlockSpec((1, H), lambda i: (0, 0)),
            pl.BlockSpec((H, E), lambda i: (0, 0)),
            pl.BlockSpec((1, E), lambda i: (0, 0)),
        ],
        out_specs=pl.BlockSpec((S, E), lambda i: (i, 0)),
        out_shape=jax.ShapeDtypeStruct((B, E), jnp.float32),
        compiler_params=pltpu.CompilerParams(
            dimension_semantics=("arbitrary",),
        ),
    )(emb, par, W1, b1.reshape(1, H), W2, b2.reshape(1, E))


def kernel(x, table, W1, b1, W2, b2):
    B, L = x.shape
    V, D = table.shape
    H = W1.shape[1]
    E = W2.shape[1]
    NW = 32

    xi = x.astype(jnp.int32)
    idx_all = (xi >> 1).T.reshape(NW, -1)
    par = (xi & 1).T.reshape(L, B, 1).astype(jnp.float32)

    table2 = table.reshape(V // 2, 2 * D)
    emb = _sc_gather(table2, idx_all, B * L, NW)
    return _tc_mlp(
        emb.reshape(L, B, 2 * D), par, W1, b1, W2, b2, B, L, H, E, S=256
    )
